# Initial kernel scaffold; baseline (speedup 1.0000x reference)
import jax
import jax.numpy as jnp
from jax.experimental import pallas as pl


def kernel(x, edge_index, batch, W1, as1, ad1, b1, W2, as2, ad2, b2, Wm1, bm1, Wm2, bm2):
    raise NotImplementedError("write your pallas kernel here")


# speedup vs baseline: 16.9445x; 16.9445x over previous
import functools
import jax
import jax.numpy as jnp
from jax import lax
from jax.experimental import pallas as pl
from jax.experimental.pallas import tpu as pltpu
from jax.experimental.pallas import tpu_sc as plsc

N = 10000
D = 256
H = 4
C = 256
F = H * C
NG = 64
NCLS = 10
KCH = 8
CW = 128
RB = 1000
GRID = N // RB
ASW = 16

E_RAW = 160000
E_TOT = E_RAW + N
NW = 32
NSC = 16
EP = 172032
EW = EP // NW
NCHUNK = EW // 128
EW3 = EP // NSC
NCHUNK3 = EW3 // 128
NPAIR = EW3 // 128
_PIECES64 = tuple((i * 64, 64) for i in range(9)) + ((576, 48),)
TR = 624

_MESH = plsc.VectorSubcoreMesh(core_axis_name="c", subcore_axis_name="s",
                               num_cores=2, num_subcores=NSC)


def _head_mask():
    r = lax.broadcasted_iota(jnp.int32, (F, H), 0) // C
    c = lax.broadcasted_iota(jnp.int32, (F, H), 1)
    return (r == c).astype(jnp.float32)


def _dense1_body(x_ref, w_ref, asf_ref, adf_ref, h_ref, as_ref, ad_ref):
    xb = x_ref[...]
    h = jnp.dot(xb, w_ref[...], preferred_element_type=jnp.float32)
    m = _head_mask()
    as_ref[...] = jnp.dot(h * asf_ref[...], m, preferred_element_type=jnp.float32)
    ad_ref[...] = jnp.dot(h * adf_ref[...], m, preferred_element_type=jnp.float32)
    for k in range(KCH):
        h_ref[k] = h[:, k * CW:(k + 1) * CW]


def _dense1(x, W, asf, adf):
    return pl.pallas_call(
        _dense1_body,
        grid=(GRID,),
        in_specs=[
            pl.BlockSpec((RB, D), lambda i: (i, 0)),
            pl.BlockSpec((D, F), lambda i: (0, 0)),
            pl.BlockSpec((1, F), lambda i: (0, 0)),
            pl.BlockSpec((1, F), lambda i: (0, 0)),
        ],
        out_specs=[
            pl.BlockSpec((KCH, RB, CW), lambda i: (0, i, 0)),
            pl.BlockSpec((RB, H), lambda i: (i, 0)),
            pl.BlockSpec((RB, H), lambda i: (i, 0)),
        ],
        out_shape=[
            jax.ShapeDtypeStruct((KCH, N, CW), jnp.float32),
            jax.ShapeDtypeStruct((N, H), jnp.float32),
            jax.ShapeDtypeStruct((N, H), jnp.float32),
        ],
    )(x, W, asf, adf)


def _dense2_body(acc_ref, asum_ref, b_ref, w_ref, asf_ref, adf_ref,
                 h_ref, as_ref, ad_ref):
    h = jnp.zeros((RB, F), jnp.float32)
    for k in range(KCH):
        hd = k // 2
        denom = asum_ref[0, :, hd] + asum_ref[1, :, hd] + 1e-16
        o = acc_ref[k] / denom[:, None] + b_ref[0, k * CW:(k + 1) * CW]
        h = h + jnp.dot(o, w_ref[k], preferred_element_type=jnp.float32)
    m = _head_mask()
    as_ref[...] = jnp.dot(h * asf_ref[...], m, preferred_element_type=jnp.float32)
    ad_ref[...] = jnp.dot(h * adf_ref[...], m, preferred_element_type=jnp.float32)
    for k in range(KCH):
        h_ref[k] = h[:, k * CW:(k + 1) * CW]


def _dense2(acc, asum8, b, Wr, asf, adf):
    return pl.pallas_call(
        _dense2_body,
        grid=(GRID,),
        in_specs=[
            pl.BlockSpec((KCH, RB, CW), lambda i: (0, i, 0)),
            pl.BlockSpec((2, RB, ASW), lambda i: (0, i, 0)),
            pl.BlockSpec((1, F), lambda i: (0, 0)),
            pl.BlockSpec((KCH, CW, F), lambda i: (0, 0, 0)),
            pl.BlockSpec((1, F), lambda i: (0, 0)),
            pl.BlockSpec((1, F), lambda i: (0, 0)),
        ],
        out_specs=[
            pl.BlockSpec((KCH, RB, CW), lambda i: (0, i, 0)),
            pl.BlockSpec((RB, H), lambda i: (i, 0)),
            pl.BlockSpec((RB, H), lambda i: (i, 0)),
        ],
        out_shape=[
            jax.ShapeDtypeStruct((KCH, N, CW), jnp.float32),
            jax.ShapeDtypeStruct((N, H), jnp.float32),
            jax.ShapeDtypeStruct((N, H), jnp.float32),
        ],
    )(acc, asum8, b, Wr, asf, adf)


def _head_body(acc_ref, asum_ref, b_ref, batch_ref, wm1_ref, bm1_ref,
               wm2_ref, bm2_ref, out_ref, gacc):
    i = pl.program_id(0)

    @pl.when(i == 0)
    def _():
        gacc[...] = jnp.zeros_like(gacc)

    onehot = (lax.broadcasted_iota(jnp.int32, (NG, RB), 0)
              == batch_ref[0]).astype(jnp.float32)
    t = jnp.zeros((NG, 32), jnp.float32)
    for k in range(KCH):
        hd = k // 2
        denom = asum_ref[0, :, hd] + asum_ref[1, :, hd] + 1e-16
        o = acc_ref[k] / denom[:, None] + b_ref[0, k * CW:(k + 1) * CW]
        p = jnp.dot(onehot, o, preferred_element_type=jnp.float32)
        t = t + jnp.dot(p, wm1_ref[k], preferred_element_type=jnp.float32)
    gacc[...] += t

    @pl.when(i == GRID - 1)
    def _():
        g = jax.nn.relu(gacc[...] + bm1_ref[...])
        out_ref[...] = jnp.dot(g, wm2_ref[...],
                               preferred_element_type=jnp.float32) + bm2_ref[...]


def _head(acc, asum8, b, batch2, Wm1r, bm1, Wm2, bm2):
    return pl.pallas_call(
        _head_body,
        grid=(GRID,),
        in_specs=[
            pl.BlockSpec((KCH, RB, CW), lambda i: (0, i, 0)),
            pl.BlockSpec((2, RB, ASW), lambda i: (0, i, 0)),
            pl.BlockSpec((1, F), lambda i: (0, 0)),
            pl.BlockSpec((1, 1, RB), lambda i: (i, 0, 0)),
            pl.BlockSpec((KCH, CW, 32), lambda i: (0, 0, 0)),
            pl.BlockSpec((1, 32), lambda i: (0, 0)),
            pl.BlockSpec((32, NCLS), lambda i: (0, 0)),
            pl.BlockSpec((1, NCLS), lambda i: (0, 0)),
        ],
        out_specs=pl.BlockSpec((NG, NCLS), lambda i: (0, 0)),
        out_shape=jax.ShapeDtypeStruct((NG, NCLS), jnp.float32),
        scratch_shapes=[pltpu.VMEM((NG, 32), jnp.float32)],
    )(acc, asum8, b, batch2, Wm1r, bm1, Wm2, bm2)


_CP_SC = pltpu.CompilerParams(needs_layout_passes=False,
                              use_tc_tiling_on_sc=False)
_PIECES = ((0, 128), (128, 128), (256, 128), (384, 128), (512, 112))
_TAIL = N - NSC * TR


def _iota16():
    return lax.iota(jnp.int32, 16)


def _edge_sc_body(src_h, dst_h, as_h, ad_h, z16_h,
                  wt_h, asum_h,
                  t_as, t_ad, srcv, dstv, wtile, wflat, w16, accS):
    cid = lax.axis_index("c")
    sid = lax.axis_index("s")
    wid = cid * NSC + sid
    base = wid * EW
    rows0 = sid * TR

    pltpu.sync_copy(as_h, t_as)
    pltpu.sync_copy(ad_h, t_ad)
    pltpu.sync_copy(z16_h.at[pl.ds(0, 128)], w16)
    for p0, sz in _PIECES:
        pltpu.sync_copy(w16.at[pl.ds(0, sz)], accS.at[pl.ds(rows0 + p0, sz)])

    @pl.when(sid == NSC - 1)
    def _():
        pltpu.sync_copy(w16.at[pl.ds(0, _TAIL)],
                        accS.at[pl.ds(NSC * TR, _TAIL)])
    plsc.subcore_barrier()
    maskf = jnp.where(_iota16() < H, 1.0, 0.0).astype(jnp.float32)

    def chunk(ci):
        off = pl.multiple_of(base + ci * 128, 8)
        pltpu.sync_copy(src_h.at[pl.ds(off, 128)], srcv)
        pltpu.sync_copy(dst_h.at[pl.ds(off, 128)], dstv)
        for j in range(8):
            s16 = srcv[pl.ds(j * 16, 16)] * H
            d16 = dstv[pl.ds(j * 16, 16)] * H
            eid = off + j * 16 + _iota16()
            valid = eid < E_TOT
            for h in range(H):
                a_s = plsc.load_gather(t_as, [s16 + h])
                a_dd = plsc.load_gather(t_ad, [d16 + h])
                a_sd = plsc.load_gather(t_as, [d16 + h])
                al = a_s + a_dd
                al = jnp.maximum(al, 0.2 * al)
                cc = a_sd + a_dd
                cc = jnp.maximum(cc, 0.2 * cc)
                w = jnp.exp(al - cc)
                w = jnp.where(valid, w, 0.0)
                wtile[h, pl.ds(ci * 128 + j * 16, 16)] = w
                wflat[pl.ds(h * 128 + j * 16, 16)] = w
        for e in range(128):
            gidx = jnp.minimum(e + 128 * _iota16(), H * 128 - 1)
            row = plsc.load_gather(wflat, [gidx]) * maskf
            w16[e, pl.ds(0, ASW)] = row
        pltpu.sync_copy(w16, accS.at[dstv], add=True)

    pl.loop(0, NCHUNK)(chunk)

    plsc.subcore_barrier()
    for p0, sz in _PIECES:
        pltpu.sync_copy(accS.at[pl.ds(rows0 + p0, sz)], w16.at[pl.ds(0, sz)])
        pltpu.sync_copy(w16.at[pl.ds(0, sz)],
                        asum_h.at[cid, pl.ds(rows0 + p0, sz)])

    @pl.when(sid == NSC - 1)
    def _():
        pltpu.sync_copy(accS.at[pl.ds(NSC * TR, _TAIL)],
                        w16.at[pl.ds(0, _TAIL)])
        pltpu.sync_copy(w16.at[pl.ds(0, _TAIL)],
                        asum_h.at[cid, pl.ds(NSC * TR, _TAIL)])

    for h in range(H):
        pltpu.sync_copy(wtile.at[h],
                        wt_h.at[pl.ds(h * EP + wid * EW, EW)])


_EDGE_KERNEL = pl.kernel(
    _edge_sc_body,
    out_type=[
        jax.ShapeDtypeStruct((H * EP,), jnp.float32),
        jax.ShapeDtypeStruct((2, N, ASW), jnp.float32),
    ],
    mesh=_MESH,
    compiler_params=_CP_SC,
    scratch_types=[
        pltpu.VMEM((N * H,), jnp.float32),
        pltpu.VMEM((N * H,), jnp.float32),
        pltpu.VMEM((128,), jnp.int32),
        pltpu.VMEM((128,), jnp.int32),
        pltpu.VMEM((H, EW), jnp.float32),
        pltpu.VMEM((H * 128,), jnp.float32),
        pltpu.VMEM((128, ASW), jnp.float32),
        pltpu.VMEM_SHARED((N, ASW), jnp.float32),
    ],
)


def _edge_sc(srcp, dstp, a_s, a_d, z16):
    return _EDGE_KERNEL(srcp, dstp, a_s.reshape(N * H), a_d.reshape(N * H),
                        z16)


def _spmm_sc_body(src_h, dst_h, hf_h, wt_h, z128_h,
                  acc_h,
                  srcall, dstall, ia, ib, da, db, wtv,
                  rowsa, rowsb, accS, sema, semb):
    cid = lax.axis_index("c")
    sid = lax.axis_index("s")
    base = sid * EW3
    rows0 = sid * TR

    def build_idx(c, idxbuf, dstbuf, koff):
        for j in range(4):
            sl = pl.ds(j * 16, 16)
            idxbuf[sl] = srcall[pl.ds(c * 64 + j * 16, 16)] + koff
            dstbuf[sl] = dstall[pl.ds(c * 64 + j * 16, 16)]

    def scale(c, rows):
        for g in range(4):
            wvec = wtv[pl.ds(c * 64 + g * 16, 16)]
            for t in range(16):
                e = g * 16 + t
                bc = jnp.full((16,), wvec[t], jnp.float32)
                for j in range(8):
                    sl = pl.ds(j * 16, 16)
                    rows[e, sl] = rows[e, sl] * bc

    pltpu.sync_copy(src_h.at[pl.ds(pl.multiple_of(base, 8), EW3)], srcall)
    pltpu.sync_copy(dst_h.at[pl.ds(pl.multiple_of(base, 8), EW3)], dstall)

    for kk in range(KCH // 2):
        k = cid * (KCH // 2) + kk
        hd = k // 2
        koff = k * N
        pltpu.sync_copy(z128_h.at[pl.ds(0, 64)], rowsa)
        for p0, sz in _PIECES64:
            pltpu.sync_copy(rowsa.at[pl.ds(0, sz)],
                            accS.at[pl.ds(rows0 + p0, sz)])

        @pl.when(sid == NSC - 1)
        def _():
            pltpu.sync_copy(rowsa.at[pl.ds(0, _TAIL)],
                            accS.at[pl.ds(NSC * TR, _TAIL)])
        woff = pl.multiple_of(hd * EP + base, 8)
        pltpu.sync_copy(wt_h.at[pl.ds(woff, EW3)], wtv)
        plsc.subcore_barrier()

        build_idx(0, ia, da, koff)
        pltpu.async_copy(hf_h.at[ia], rowsa, sema).wait()

        def pair(p):
            ca = 2 * p
            cb = 2 * p + 1
            build_idx(cb, ib, db, koff)
            cpb = pltpu.async_copy(hf_h.at[ib], rowsb, semb)
            scale(ca, rowsa)
            pltpu.sync_copy(rowsa, accS.at[da], add=True)
            cpb.wait()

            @pl.when(p < NPAIR - 1)
            def _():
                build_idx(ca + 2, ia, da, koff)

            cpa = pltpu.async_copy(hf_h.at[ia], rowsa, sema)
            scale(cb, rowsb)
            pltpu.sync_copy(rowsb, accS.at[db], add=True)
            cpa.wait()

        pl.loop(0, NPAIR)(pair)

        plsc.subcore_barrier()
        for p0, sz in _PIECES64:
            pltpu.sync_copy(accS.at[pl.ds(rows0 + p0, sz)],
                            rowsa.at[pl.ds(0, sz)])
            pltpu.sync_copy(rowsa.at[pl.ds(0, sz)],
                            acc_h.at[pl.ds(k * N + rows0 + p0, sz)])

        @pl.when(sid == NSC - 1)
        def _():
            pltpu.sync_copy(accS.at[pl.ds(NSC * TR, _TAIL)],
                            rowsa.at[pl.ds(0, _TAIL)])
            pltpu.sync_copy(rowsa.at[pl.ds(0, _TAIL)],
                            acc_h.at[pl.ds(k * N + NSC * TR, _TAIL)])

        plsc.subcore_barrier()


_SPMM_KERNEL = pl.kernel(
    _spmm_sc_body,
    out_type=jax.ShapeDtypeStruct((KCH * N, CW), jnp.float32),
    mesh=_MESH,
    compiler_params=_CP_SC,
    scratch_types=[
        pltpu.VMEM((EW3,), jnp.int32),
        pltpu.VMEM((EW3,), jnp.int32),
        pltpu.VMEM((64,), jnp.int32),
        pltpu.VMEM((64,), jnp.int32),
        pltpu.VMEM((64,), jnp.int32),
        pltpu.VMEM((64,), jnp.int32),
        pltpu.VMEM((EW3,), jnp.float32),
        pltpu.VMEM((64, CW), jnp.float32),
        pltpu.VMEM((64, CW), jnp.float32),
        pltpu.VMEM_SHARED((N, CW), jnp.float32),
        pltpu.SemaphoreType.DMA,
        pltpu.SemaphoreType.DMA,
    ],
)


def _spmm_sc(srcp, dstp, h_flat, wt_flat, z128):
    return _SPMM_KERNEL(srcp, dstp, h_flat, wt_flat, z128)


def _layer_sparse(srcp, dstp, a_s, a_d, hc, z16, z128):
    wt_flat, asum = _edge_sc(srcp, dstp, a_s, a_d, z16)
    acc_flat = _spmm_sc(srcp, dstp, hc.reshape(KCH * N, CW), wt_flat, z128)
    return acc_flat.reshape(KCH, N, CW), asum


def kernel(x, edge_index, batch, W1, as1, ad1, b1, W2, as2, ad2, b2,
           Wm1, bm1, Wm2, bm2):
    loop = jnp.arange(N, dtype=edge_index.dtype)
    pad = jnp.zeros((EP - E_TOT,), edge_index.dtype)
    srcp = jnp.concatenate([edge_index[0], loop, pad])
    dstp = jnp.concatenate([edge_index[1], loop, pad])
    z16 = jnp.zeros((TR, ASW), jnp.float32)
    z128 = jnp.zeros((TR, CW), jnp.float32)

    as1f = as1.reshape(1, F)
    ad1f = ad1.reshape(1, F)
    as2f = as2.reshape(1, F)
    ad2f = ad2.reshape(1, F)
    W2r = W2.reshape(KCH, CW, F)
    Wm1r = Wm1.reshape(KCH, CW, 32)
    b1r = b1.reshape(1, F)
    b2r = b2.reshape(1, F)
    bm1r = bm1.reshape(1, 32)
    bm2r = bm2.reshape(1, NCLS)
    batch2 = batch.reshape(GRID, 1, RB)

    h1c, as_1, ad_1 = _dense1(x, W1, as1f, ad1f)
    acc1, asum1 = _layer_sparse(srcp, dstp, as_1, ad_1, h1c, z16, z128)
    h2c, as_2, ad_2 = _dense2(acc1, asum1, b1r, W2r, as2f, ad2f)
    acc2, asum2 = _layer_sparse(srcp, dstp, as_2, ad_2, h2c, z16, z128)
    return _head(acc2, asum2, b2r, batch2, Wm1r, bm1r, Wm2, bm2r)
